# Initial kernel scaffold; baseline (speedup 1.0000x reference)
import jax
import jax.numpy as jnp
from jax.experimental import pallas as pl


def kernel(x, Wm, bm, Wh, bh, Wz, bz, Wr, br, Wi, bi):
    raise NotImplementedError("write your pallas kernel here")


# speedup vs baseline: 2.8555x; 2.8555x over previous
import jax
import jax.numpy as jnp
from jax.experimental import pallas as pl
from jax.experimental.pallas import tpu as pltpu


def _gru_body(T, BB, nb,
              xt_ref, wh_ref, wm_ref, wzu_ref, wru_ref, wiu_ref,
              wzh_ref, wrh_ref, wih_ref, bh_ref, bz_ref, br_ref, bi_ref,
              out_ref, h_scr, gz_scr, gr_scr, gi_scr):
    j = pl.program_id(1)

    @pl.when(j == 0)
    def _init():
        h_scr[...] = (
            jnp.dot(xt_ref[0], wh_ref[...], preferred_element_type=jnp.float32)
            + bh_ref[...]
        )

    xb = xt_ref[...].reshape(T * BB, xt_ref.shape[2])
    u = jnp.dot(xb, wm_ref[...], preferred_element_type=jnp.float32)
    u_bf = u.astype(jnp.bfloat16)
    gz_scr[...] = (
        jnp.dot(u_bf, wzu_ref[...], preferred_element_type=jnp.float32)
        + bz_ref[...]
    )
    gr_scr[...] = (
        jnp.dot(u_bf, wru_ref[...], preferred_element_type=jnp.float32)
        + br_ref[...]
    )
    gi_scr[...] = (
        jnp.dot(u_bf, wiu_ref[...], preferred_element_type=jnp.float32)
        + bi_ref[...]
    )

    h = h_scr[...]
    for t in range(T):
        hb = h.astype(jnp.bfloat16)
        z = jax.nn.sigmoid(
            gz_scr[t * BB:(t + 1) * BB, :]
            + jnp.dot(hb, wzh_ref[...], preferred_element_type=jnp.float32)
        )
        r = jax.nn.sigmoid(
            gr_scr[t * BB:(t + 1) * BB, :]
            + jnp.dot(hb, wrh_ref[...], preferred_element_type=jnp.float32)
        )
        hp = jnp.tanh(
            gi_scr[t * BB:(t + 1) * BB, :]
            + jnp.dot((r * h).astype(jnp.bfloat16), wih_ref[...],
                      preferred_element_type=jnp.float32)
        )
        h = (1.0 - z) * h + z * hp
    h_scr[...] = h

    @pl.when(j == nb - 1)
    def _fin():
        out_ref[...] = h


def kernel(x, Wm, bm, Wh, bh, Wz, bz, Wr, br, Wi, bi):
    B, S, D = x.shape
    H = Wm.shape[0]
    T = 16
    NB = S // T
    NCORES = 2
    BB = B // NCORES

    bf = jnp.bfloat16
    xt = jnp.swapaxes(x, 0, 1).astype(bf)
    wh = Wh.T.astype(bf)
    wm = Wm.T.astype(bf)
    wzu = Wz[:, :H].T.astype(bf)
    wru = Wr[:, :H].T.astype(bf)
    wiu = Wi[:, :H].T.astype(bf)
    wzh = Wz[:, H:].T.astype(bf)
    wrh = Wr[:, H:].T.astype(bf)
    wih = Wi[:, H:].T.astype(bf)
    bz_eff = (bz + bm @ Wz[:, :H].T).reshape(1, H)
    br_eff = (br + bm @ Wr[:, :H].T).reshape(1, H)
    bi_eff = (bi + bm @ Wi[:, :H].T).reshape(1, H)
    bh2 = bh.reshape(1, H)

    full = lambda a: pl.BlockSpec(a.shape, lambda i, j: (0,) * a.ndim)

    out = pl.pallas_call(
        lambda *refs: _gru_body(T, BB, NB, *refs),
        grid=(NCORES, NB),
        in_specs=[
            pl.BlockSpec((T, BB, D), lambda i, j: (j, i, 0)),
            full(wh), full(wm),
            full(wzu), full(wru), full(wiu),
            full(wzh), full(wrh), full(wih),
            full(bh2), full(bz_eff), full(br_eff), full(bi_eff),
        ],
        out_specs=pl.BlockSpec((BB, H), lambda i, j: (i, 0)),
        out_shape=jax.ShapeDtypeStruct((B, H), jnp.float32),
        scratch_shapes=[
            pltpu.VMEM((BB, H), jnp.float32),
            pltpu.VMEM((T * BB, H), jnp.float32),
            pltpu.VMEM((T * BB, H), jnp.float32),
            pltpu.VMEM((T * BB, H), jnp.float32),
        ],
        compiler_params=pltpu.CompilerParams(
            dimension_semantics=("parallel", "arbitrary"),
            vmem_limit_bytes=56 * 1024 * 1024,
        ),
    )(xt, wh, wm, wzu, wru, wiu, wzh, wrh, wih, bh2, bz_eff, br_eff, bi_eff)

    return out[:, None, :]
